# baseline (speedup 1.0000x reference)
import jax
import jax.numpy as jnp
from jax import lax
from jax.experimental import pallas as pl
from jax.experimental.pallas import tpu as pltpu


def _fused_kernel(x_ref, w_ref, b_ref, o_ref):
    x = x_ref[...]
    y = lax.dot_general(
        x.astype(jnp.bfloat16),
        w_ref[...],
        dimension_numbers=(((1,), (0,)), ((), ())),
        preferred_element_type=jnp.float32,
    )
    o_ref[...] = x + y + b_ref[...]


def kernel(x2d, w_out_in, b):
    M, H = x2d.shape
    TM = 1024
    m_pad = pl.cdiv(M, TM) * TM
    x_in = x2d if m_pad == M else jnp.pad(x2d, ((0, m_pad - M), (0, 0)))
    m_tiles = m_pad // TM

    w_bf16 = w_out_in.T.astype(jnp.bfloat16)

    out = pl.pallas_call(
        _fused_kernel,
        out_shape=jax.ShapeDtypeStruct((m_pad, H), x2d.dtype),
        grid=(m_tiles,),
        in_specs=[
            pl.BlockSpec((TM, H), lambda i: (i, 0)),
            pl.BlockSpec((H, H), lambda i: (0, 0)),
            pl.BlockSpec((1, H), lambda i: (0, 0)),
        ],
        out_specs=pl.BlockSpec((TM, H), lambda i: (i, 0)),
        compiler_params=pltpu.CompilerParams(
            dimension_semantics=("arbitrary",),
            vmem_limit_bytes=60 * 1024 * 1024,
        ),
        cost_estimate=pl.CostEstimate(
            flops=2 * m_pad * H * H,
            transcendentals=0,
            bytes_accessed=2 * m_pad * H * 4 + w_bf16.nbytes + b.nbytes,
        ),
    )(x_in, w_bf16, b.reshape(1, H))

    return out[:M] if m_pad != M else out


# speedup vs baseline: 1.2709x; 1.2709x over previous
import jax
import jax.numpy as jnp
from jax import lax
from jax.experimental import pallas as pl
from jax.experimental.pallas import tpu as pltpu


def _fused_kernel(x_ref, w_ref, b_ref, o_ref):
    x = x_ref[...]
    y = lax.dot_general(
        x,
        w_ref[...],
        dimension_numbers=(((1,), (1,)), ((), ())),
        preferred_element_type=jnp.float32,
    )
    o_ref[...] = x + y + b_ref[...]


def kernel(x2d, w_out_in, b):
    M, H = x2d.shape
    TM = 512
    m_pad = pl.cdiv(M, TM) * TM
    x_in = x2d if m_pad == M else jnp.pad(x2d, ((0, m_pad - M), (0, 0)))
    m_tiles = m_pad // TM

    out = pl.pallas_call(
        _fused_kernel,
        out_shape=jax.ShapeDtypeStruct((m_pad, H), x2d.dtype),
        grid=(m_tiles,),
        in_specs=[
            pl.BlockSpec((TM, H), lambda i: (i, 0)),
            pl.BlockSpec((H, H), lambda i: (0, 0)),
            pl.BlockSpec((1, H), lambda i: (0, 0)),
        ],
        out_specs=pl.BlockSpec((TM, H), lambda i: (i, 0)),
        compiler_params=pltpu.CompilerParams(
            dimension_semantics=("arbitrary",),
            vmem_limit_bytes=60 * 1024 * 1024,
        ),
        cost_estimate=pl.CostEstimate(
            flops=2 * m_pad * H * H,
            transcendentals=0,
            bytes_accessed=2 * m_pad * H * 4 + w_out_in.nbytes + b.nbytes,
        ),
    )(x_in, w_out_in, b.reshape(1, H))

    return out[:M] if m_pad != M else out
